# Initial kernel scaffold; baseline (speedup 1.0000x reference)
import functools

import jax
import jax.numpy as jnp
import numpy as np
from jax import lax
from jax.experimental import pallas as pl
from jax.experimental.pallas import tpu as pltpu
from jax.experimental.pallas import tpu_sc as plsc

_NUM_WORKERS = 32
_LANES = 16


def _position_encoding(max_len: int, d_model: int) -> np.ndarray:
    angle_rates = 10000.0 ** (2.0 * (np.arange(d_model, dtype=np.float64) / d_model))
    angle = np.arange(max_len, dtype=np.float64)[:, None] / angle_rates
    values = np.stack([np.sin(angle[:, 0::2]), np.cos(angle[:, 1::2])], axis=2)
    return values.reshape(max_len, -1).astype(np.float32)


@functools.lru_cache(maxsize=None)
def _build(n_rows: int, seq_len: int, d_model: int):
    assert d_model % _LANES == 0
    per_w = n_rows // _NUM_WORKERS
    assert per_w * _NUM_WORKERS == n_rows
    assert per_w % seq_len == 0
    chunk = seq_len
    while per_w % (2 * chunk) == 0 and 2 * (2 * chunk) * (d_model + 1) * 4 < 420_000:
        chunk *= 2
    n_chunks = per_w // chunk
    reps = chunk // seq_len
    n_seg = d_model // _LANES

    mesh = plsc.VectorSubcoreMesh(core_axis_name="c", subcore_axis_name="s")

    def body(seq_hbm, table_hbm, pe_hbm, out_hbm,
             idx0, idx1, rows0, rows1, pe_v, gsem0, gsem1, ssem0, ssem1):
        wid = lax.axis_index("s") * 2 + lax.axis_index("c")
        base = wid * per_w
        idx = (idx0, idx1)
        rows = (rows0, rows1)
        gsem = (gsem0, gsem1)
        ssem = (ssem0, ssem1)

        pltpu.sync_copy(pe_hbm, pe_v)

        def fused_scale_pe(rows_ref):
            def lbody(l, carry):
                pes = [pe_v[l, pl.ds(s * _LANES, _LANES)] for s in range(n_seg)]
                for rep in range(reps):
                    r = rep * seq_len + l
                    for s in range(n_seg):
                        sl = pl.ds(s * _LANES, _LANES)
                        rows_ref[r, sl] = rows_ref[r, sl] * 8.0 + pes[s]
                return carry
            lax.fori_loop(0, seq_len, lbody, 0)

        pltpu.sync_copy(seq_hbm.at[pl.ds(base, chunk)], idx0)
        gathers = {0: pltpu.async_copy(table_hbm.at[idx0], rows0, gsem0)}
        stores = {}
        for c in range(n_chunks):
            b = c & 1
            nb = 1 - b
            if c + 1 < n_chunks:
                pltpu.sync_copy(
                    seq_hbm.at[pl.ds(base + (c + 1) * chunk, chunk)], idx[nb])
                if c >= 1:
                    stores[c - 1].wait()
                gathers[c + 1] = pltpu.async_copy(
                    table_hbm.at[idx[nb]], rows[nb], gsem[nb])
            gathers[c].wait()
            fused_scale_pe(rows[b])
            stores[c] = pltpu.async_copy(
                rows[b], out_hbm.at[pl.ds(base + c * chunk, chunk)], ssem[b])
        stores[n_chunks - 2].wait()
        stores[n_chunks - 1].wait()

    return pl.kernel(
        body,
        out_type=jax.ShapeDtypeStruct((n_rows, d_model), jnp.float32),
        mesh=mesh,
        scratch_types=[
            pltpu.VMEM((chunk,), jnp.int32),
            pltpu.VMEM((chunk,), jnp.int32),
            pltpu.VMEM((chunk, d_model), jnp.float32),
            pltpu.VMEM((chunk, d_model), jnp.float32),
            pltpu.VMEM((seq_len, d_model), jnp.float32),
            pltpu.SemaphoreType.DMA,
            pltpu.SemaphoreType.DMA,
            pltpu.SemaphoreType.DMA,
            pltpu.SemaphoreType.DMA,
        ],
    )


def kernel(sequences, emb_table):
    batch, seq_len = sequences.shape
    d_model = emb_table.shape[1]
    n_rows = batch * seq_len
    pe = jnp.asarray(_position_encoding(seq_len, d_model))
    run = _build(n_rows, seq_len, d_model)
    out = run(sequences.reshape(n_rows), emb_table, pe)
    return out.reshape(batch, seq_len, d_model)


# speedup vs baseline: 3.2207x; 3.2207x over previous
import functools

import jax
import jax.numpy as jnp
import numpy as np
from jax import lax
from jax.experimental import pallas as pl
from jax.experimental.pallas import tpu as pltpu
from jax.experimental.pallas import tpu_sc as plsc

_NUM_WORKERS = 32
_LANES = 16


def _position_encoding(max_len: int, d_model: int) -> np.ndarray:
    angle_rates = 10000.0 ** (2.0 * (np.arange(d_model, dtype=np.float64) / d_model))
    angle = np.arange(max_len, dtype=np.float64)[:, None] / angle_rates
    values = np.stack([np.sin(angle[:, 0::2]), np.cos(angle[:, 1::2])], axis=2)
    return values.reshape(max_len, -1).astype(np.float32)


@functools.lru_cache(maxsize=None)
def _build(n_rows: int, seq_len: int, d_model: int):
    assert d_model % _LANES == 0
    per_w = n_rows // _NUM_WORKERS
    assert per_w * _NUM_WORKERS == n_rows
    assert per_w % seq_len == 0
    chunk = seq_len
    while per_w % (2 * chunk) == 0 and 2 * (2 * chunk) * (d_model + 1) * 4 < 420_000:
        chunk *= 2
    n_chunks = per_w // chunk
    reps = chunk // seq_len
    n_seg = d_model // _LANES

    mesh = plsc.VectorSubcoreMesh(
        core_axis_name="c", subcore_axis_name="s", num_cores=2, num_subcores=16)

    def body(seq_hbm, table_hbm, pe_hbm, out_hbm,
             idx0, idx1, rows0, rows1, pe_v, gsem0, gsem1, ssem0, ssem1):
        wid = lax.axis_index("s") * 2 + lax.axis_index("c")
        base = wid * per_w
        idx = (idx0, idx1)
        rows = (rows0, rows1)
        gsem = (gsem0, gsem1)
        ssem = (ssem0, ssem1)

        pltpu.sync_copy(pe_hbm, pe_v)

        def fused_scale_pe(rows_ref):
            def lbody(l, carry):
                pes = [pe_v[l, pl.ds(s * _LANES, _LANES)] for s in range(n_seg)]
                for rep in range(reps):
                    r = rep * seq_len + l
                    for s in range(n_seg):
                        sl = pl.ds(s * _LANES, _LANES)
                        rows_ref[r, sl] = rows_ref[r, sl] * 8.0 + pes[s]
                return carry
            lax.fori_loop(0, seq_len, lbody, 0)

        pltpu.sync_copy(seq_hbm.at[pl.ds(base, chunk)], idx0)
        gathers = {0: pltpu.async_copy(table_hbm.at[idx0], rows0, gsem0)}
        stores = {}
        for c in range(n_chunks):
            b = c & 1
            nb = 1 - b
            if c + 1 < n_chunks:
                pltpu.sync_copy(
                    seq_hbm.at[pl.ds(base + (c + 1) * chunk, chunk)], idx[nb])
                if c >= 1:
                    stores[c - 1].wait()
                gathers[c + 1] = pltpu.async_copy(
                    table_hbm.at[idx[nb]], rows[nb], gsem[nb])
            gathers[c].wait()
            fused_scale_pe(rows[b])
            stores[c] = pltpu.async_copy(
                rows[b], out_hbm.at[pl.ds(base + c * chunk, chunk)], ssem[b])
        stores[n_chunks - 2].wait()
        stores[n_chunks - 1].wait()

    return pl.kernel(
        body,
        out_type=jax.ShapeDtypeStruct((n_rows, d_model), jnp.float32),
        mesh=mesh,
        compiler_params=pltpu.CompilerParams(use_tc_tiling_on_sc=False),
        scratch_types=[
            pltpu.VMEM((chunk,), jnp.int32),
            pltpu.VMEM((chunk,), jnp.int32),
            pltpu.VMEM((chunk, d_model), jnp.float32),
            pltpu.VMEM((chunk, d_model), jnp.float32),
            pltpu.VMEM((seq_len, d_model), jnp.float32),
            pltpu.SemaphoreType.DMA,
            pltpu.SemaphoreType.DMA,
            pltpu.SemaphoreType.DMA,
            pltpu.SemaphoreType.DMA,
        ],
    )


def kernel(sequences, emb_table):
    batch, seq_len = sequences.shape
    d_model = emb_table.shape[1]
    n_rows = batch * seq_len
    pe = jnp.asarray(_position_encoding(seq_len, d_model))
    run = _build(n_rows, seq_len, d_model)
    out = run(sequences.reshape(n_rows), emb_table, pe)
    return out.reshape(batch, seq_len, d_model)
